# Initial kernel scaffold; baseline (speedup 1.0000x reference)
import jax
import jax.numpy as jnp
from jax.experimental import pallas as pl


def kernel(user_ids, item_ids, user_table, item_table, W_user, W_item):
    raise NotImplementedError("write your pallas kernel here")


# speedup vs baseline: 3.4258x; 3.4258x over previous
import jax
import jax.numpy as jnp
from jax import lax
from jax.experimental import pallas as pl
from jax.experimental.pallas import tpu as pltpu
from jax.experimental.pallas import tpu_sc as plsc

D = 32
H = 768
NC = 2
NS = 16
NW = NC * NS
CH = 128
SPG = 8
GROUP = CH * SPG


def _sc_gather(item_idx, user_idx, item_table, user_table, ni, nu):
  ipw = ni // NW
  upw = nu // NW
  igroups = ipw // GROUP
  ustreams = upw // CH
  mesh = plsc.VectorSubcoreMesh(core_axis_name="c", subcore_axis_name="s")

  def body(item_idx_ref, user_idx_ref, item_tab_ref, user_tab_ref,
           items_out, users_out, idx_i, rows_i, idx_u, rows_u, sem):
    wid = lax.axis_index("s") * NC + lax.axis_index("c")

    @pl.loop(0, igroups)
    def _items(g):
      irow = wid * (ipw // CH) + g * SPG
      pltpu.sync_copy(item_idx_ref.at[pl.ds(irow, SPG)], idx_i)
      cps = [pltpu.async_copy(item_tab_ref.at[idx_i.at[j]],
                              rows_i.at[pl.ds(j * CH, CH)], sem)
             for j in range(SPG)]
      for cp in cps:
        cp.wait()
      pltpu.sync_copy(rows_i, items_out.at[pl.ds(wid * ipw + g * GROUP, GROUP)])

    urow = wid * ustreams
    pltpu.sync_copy(user_idx_ref.at[pl.ds(urow, ustreams)], idx_u)
    cps = [pltpu.async_copy(user_tab_ref.at[idx_u.at[j]],
                            rows_u.at[pl.ds(j * CH, CH)], sem)
           for j in range(ustreams)]
    for cp in cps:
      cp.wait()
    pltpu.sync_copy(rows_u, users_out.at[pl.ds(wid * upw, upw)])

  fn = pl.kernel(
      body,
      out_type=(jax.ShapeDtypeStruct((ni, D), jnp.float32),
                jax.ShapeDtypeStruct((nu, D), jnp.float32)),
      mesh=mesh,
      compiler_params=pltpu.CompilerParams(use_tc_tiling_on_sc=False),
      scratch_types=[
          pltpu.VMEM((SPG, CH), jnp.int32),
          pltpu.VMEM((GROUP, D), jnp.float32),
          pltpu.VMEM((ustreams, CH), jnp.int32),
          pltpu.VMEM((ustreams * CH, D), jnp.float32),
          pltpu.SemaphoreType.DMA,
      ],
  )
  return fn(item_idx, user_idx, item_table, user_table)


def _project(x, w, bm):
  m = x.shape[0]

  def mm(x_ref, w_ref, o_ref):
    o_ref[...] = lax.dot_general(x_ref[...], w_ref[...],
                                 (((1,), (1,)), ((), ())),
                                 preferred_element_type=jnp.float32)

  return pl.pallas_call(
      mm,
      grid=(m // bm,),
      in_specs=[pl.BlockSpec((bm, D), lambda i: (i, 0)),
                pl.BlockSpec((H, D), lambda i: (0, 0))],
      out_specs=pl.BlockSpec((bm, H), lambda i: (i, 0)),
      out_shape=jax.ShapeDtypeStruct((m, H), jnp.float32),
  )(x, w)


def kernel(user_ids, item_ids, user_table, item_table, W_user, W_item):
  b, l = item_ids.shape
  ni = b * l
  item_idx = item_ids.reshape(ni // CH, CH)
  user_idx = user_ids.reshape(b // CH, CH)
  items_g, users_g = _sc_gather(item_idx, user_idx, item_table, user_table,
                                ni, b)
  u_proj = _project(users_g, W_user, 512)
  i_proj = _project(items_g, W_item, 1024).reshape(b, l, H)
  return (u_proj, i_proj)
